# baseline (speedup 1.0000x reference)
import functools

import jax
import jax.numpy as jnp
import numpy as np
from jax.experimental import pallas as pl
from jax.experimental.pallas import tpu as pltpu

N = 10000
F = 128
FO = 64
E = 320000
AVG_LOG = float(np.log(np.arange(4) + 1.0).sum() / 4.0)
EPS = 1e-5
ECHUNK = 8000
NEG = -3.0e38


def _bn_kernel(o_ref, g_ref, b_ref, out_ref, *, do_tanh):
    o = o_ref[...]
    mu = jnp.mean(o, axis=0, keepdims=True)
    var = jnp.mean(o * o, axis=0, keepdims=True) - mu * mu
    r = (o - mu) * jax.lax.rsqrt(var + EPS) * g_ref[...] + b_ref[...]
    if do_tanh:
        r = jnp.tanh(r)
    out_ref[...] = r


def _bn(o, gamma, beta, do_tanh):
    f = o.shape[1]
    return pl.pallas_call(
        functools.partial(_bn_kernel, do_tanh=do_tanh),
        out_shape=jax.ShapeDtypeStruct(o.shape, jnp.float32),
    )(o, gamma.reshape(1, f), beta.reshape(1, f))


def _mm_kernel(x_ref, w_ref, b_ref, o_ref):
    o_ref[...] = (
        jnp.dot(x_ref[...], w_ref[...], preferred_element_type=jnp.float32)
        + b_ref[...]
    )


def _mm(x, w, b):
    k, m = w.shape
    blk = 2000
    return pl.pallas_call(
        _mm_kernel,
        grid=(N // blk,),
        in_specs=[
            pl.BlockSpec((blk, k), lambda i: (i, 0)),
            pl.BlockSpec((k, m), lambda i: (0, 0)),
            pl.BlockSpec((1, m), lambda i: (0, 0)),
        ],
        out_specs=pl.BlockSpec((blk, m), lambda i: (i, 0)),
        out_shape=jax.ShapeDtypeStruct((x.shape[0], m), jnp.float32),
    )(x, w, b.reshape(1, m))


def _seg_kernel(eb_ref, xb_ref, S_ref, S2_ref, M_ref, cnt_ref):
    step = pl.program_id(0)

    @pl.when(step == 0)
    def _init():
        S_ref[...] = jnp.zeros_like(S_ref)
        S2_ref[...] = jnp.zeros_like(S2_ref)
        M_ref[...] = jnp.full_like(M_ref, NEG)
        cnt_ref[...] = jnp.zeros_like(cnt_ref)

    def body(i, _):
        s = eb_ref[0, 0, i]
        d = eb_ref[0, 1, i]
        row = xb_ref[pl.ds(s, 1), :]
        S_ref[pl.ds(d, 1), :] += row
        S2_ref[pl.ds(d, 1), :] += row * row
        M_ref[pl.ds(d, 1), :] = jnp.maximum(M_ref[pl.ds(d, 1), :], row)
        cnt_ref[pl.ds(d, 1), :] += 1.0
        return 0

    jax.lax.fori_loop(0, ECHUNK, body, 0)


def _seg(ei3, xb):
    shp = jax.ShapeDtypeStruct((N, F), jnp.float32)
    return pl.pallas_call(
        _seg_kernel,
        grid=(E // ECHUNK,),
        in_specs=[
            pl.BlockSpec((1, 2, ECHUNK), lambda i: (i, 0, 0),
                         memory_space=pltpu.SMEM),
            pl.BlockSpec((N, F), lambda i: (0, 0)),
        ],
        out_specs=[pl.BlockSpec((N, F), lambda i: (0, 0))] * 4,
        out_shape=[shp, shp, shp, shp],
    )(ei3, xb)


def _tail_kernel(xn_ref, xa_ref, S_ref, S2_ref, M_ref, cnt_ref, w_ref, b_ref,
                 o_ref):
    cnt = cnt_ref[...]
    cnt_c = jnp.maximum(cnt, 1.0)
    inv = 1.0 / cnt_c
    ind = (cnt > 0.0).astype(jnp.float32)
    xa = xa_ref[...]
    Sm = S_ref[...] * inv
    mean = ind * xa + Sm
    std = jnp.sqrt(jax.nn.relu(S2_ref[...] * inv - Sm * Sm) + 1e-5)
    mx = jnp.where(cnt > 0.0, xa + M_ref[...], 0.0)
    amp = jnp.log(cnt_c + 1.0) * (1.0 / AVG_LOG)
    feat = jnp.concatenate(
        [xn_ref[...], amp * mean, amp * std, amp * mx, mean, std, mx], axis=1)
    o_ref[...] = (
        jnp.dot(feat, w_ref[...], preferred_element_type=jnp.float32)
        + b_ref[...]
    )


def _tail(xn, xa, S, S2, M, cnt, Wc, bc):
    blk = 2000
    return pl.pallas_call(
        _tail_kernel,
        grid=(N // blk,),
        in_specs=[pl.BlockSpec((blk, F), lambda i: (i, 0))] * 6
        + [
            pl.BlockSpec((7 * F, FO), lambda i: (0, 0)),
            pl.BlockSpec((1, FO), lambda i: (0, 0)),
        ],
        out_specs=pl.BlockSpec((blk, FO), lambda i: (i, 0)),
        out_shape=jax.ShapeDtypeStruct((N, FO), jnp.float32),
    )(xn, xa, S, S2, M, cnt, Wc, bc.reshape(1, FO))


def _fold_weights(p):
    W = p["post_W"]
    W0 = W[:F]
    blk = [W[F + i * 4 * F: F + (i + 1) * 4 * F] for i in range(4)]
    Ws = blk[0] + blk[1] + blk[2]
    Wc = jnp.concatenate(
        [W0, Ws[:F] + Ws[F:2 * F], Ws[2 * F:3 * F], Ws[3 * F:],
         blk[3][:F] + blk[3][F:2 * F], blk[3][2 * F:3 * F], blk[3][3 * F:]],
        axis=0)
    Wc = Wc @ p["lin_W"]
    bc = p["post_b"] @ p["lin_W"] + p["lin_b"]
    return Wc, bc


def kernel(x, edge_index_p, edge_index_s, edge_index_v, params):
    xn = _bn(x, params["in_gamma"], params["in_beta"], do_tanh=False)

    Wcat = jnp.concatenate(
        [params[r]["pre_W"] for r in ("p", "s", "v")], axis=1)
    A = Wcat[:F]
    B = Wcat[F:]
    bcat = jnp.concatenate(
        [params[r]["pre_b"] for r in ("p", "s", "v")]
        + [jnp.zeros((3 * F,), jnp.float32)])
    XAB = _mm(xn, jnp.concatenate([A, B], axis=1), bcat)

    outs = []
    for k, (r, ei) in enumerate(zip(("p", "s", "v"),
                                    (edge_index_p, edge_index_s, edge_index_v))):
        p = params[r]
        XA = XAB[:, F * k: F * k + F]
        XB = XAB[:, 3 * F + F * k: 3 * F + F * k + F]
        ei3 = ei.reshape(2, E // ECHUNK, ECHUNK).transpose(1, 0, 2)
        S, S2, M, cnt = _seg(ei3, XB)
        Wc, bc = _fold_weights(p)
        o = _tail(xn, XA, S, S2, M, cnt, Wc, bc)
        outs.append(_bn(o, p["bn_gamma"], p["bn_beta"], do_tanh=True))
    return tuple(outs)
